# Initial kernel scaffold; baseline (speedup 1.0000x reference)
import jax
import jax.numpy as jnp
from jax.experimental import pallas as pl


def kernel(x, weight, bias):
    raise NotImplementedError("write your pallas kernel here")


# speedup vs baseline: 1.3564x; 1.3564x over previous
import jax
import jax.numpy as jnp
from jax.experimental import pallas as pl

NUM_EXPERTS = 64
TOP_K = 8
HIDDEN = 768
T_TOKENS = 32768
BLOCK_T = 2048


def _router_kernel(x_ref, wt_ref, bias_ref, scores_ref, idx_ref, cnt_ref):
    logits = jnp.dot(x_ref[...], wt_ref[...], preferred_element_type=jnp.float32)
    logits = logits + bias_ref[...]

    cols = jax.lax.broadcasted_iota(jnp.int32, logits.shape, 1)
    work = logits
    selected = jnp.zeros(logits.shape, dtype=jnp.bool_)
    tops = []
    idxs = []
    for _ in range(TOP_K):
        m = jnp.max(work, axis=1, keepdims=True)
        hit = jnp.min(jnp.where(work == m, cols, NUM_EXPERTS), axis=1, keepdims=True)
        is_hit = cols == hit
        selected = jnp.logical_or(selected, is_hit)
        work = jnp.where(is_hit, -jnp.inf, work)
        tops.append(m)
        idxs.append(hit)
    top = jnp.concatenate(tops, axis=1)
    tidx = jnp.concatenate(idxs, axis=1)

    e = jnp.exp(top - top[:, 0:1])
    scores_ref[...] = e / jnp.sum(e, axis=1, keepdims=True)
    idx_ref[...] = tidx

    blk_cnt = jnp.sum(selected.astype(jnp.float32), axis=0, keepdims=True)

    @pl.when(pl.program_id(0) == 0)
    def _init():
        cnt_ref[...] = jnp.zeros_like(cnt_ref)

    cnt_ref[...] += blk_cnt


def kernel(x, weight, bias):
    t = x.shape[0]
    grid = t // BLOCK_T
    wt = weight.T
    bias2 = bias.reshape(1, NUM_EXPERTS)

    scores, tidx, cnt = pl.pallas_call(
        _router_kernel,
        grid=(grid,),
        in_specs=[
            pl.BlockSpec((BLOCK_T, HIDDEN), lambda i: (i, 0)),
            pl.BlockSpec((HIDDEN, NUM_EXPERTS), lambda i: (0, 0)),
            pl.BlockSpec((1, NUM_EXPERTS), lambda i: (0, 0)),
        ],
        out_specs=[
            pl.BlockSpec((BLOCK_T, TOP_K), lambda i: (i, 0)),
            pl.BlockSpec((BLOCK_T, TOP_K), lambda i: (i, 0)),
            pl.BlockSpec((1, NUM_EXPERTS), lambda i: (0, 0)),
        ],
        out_shape=[
            jax.ShapeDtypeStruct((t, TOP_K), jnp.float32),
            jax.ShapeDtypeStruct((t, TOP_K), jnp.int32),
            jax.ShapeDtypeStruct((1, NUM_EXPERTS), jnp.float32),
        ],
    )(x, wt, bias2)
    return scores, tidx, cnt.reshape(NUM_EXPERTS)


# speedup vs baseline: 4.5803x; 3.3768x over previous
import jax
import jax.numpy as jnp
from jax.experimental import pallas as pl

NUM_EXPERTS = 64
TOP_K = 8
HIDDEN = 768
BLOCK_T = 2048

_NEG_INF = float("-inf")


def _router_kernel(x_ref, w_ref, bias_ref, scores_ref, idx_ref, cnt_ref):
    logits = jax.lax.dot_general(
        w_ref[...], x_ref[...],
        dimension_numbers=(((1,), (1,)), ((), ())),
        preferred_element_type=jnp.float32,
    )
    logits = logits + bias_ref[...]

    rows = jax.lax.broadcasted_iota(jnp.int32, logits.shape, 0)
    work = logits
    selected = jnp.zeros(logits.shape, dtype=jnp.bool_)
    tops = []
    idxs = []
    for _ in range(TOP_K):
        m = jnp.max(work, axis=0, keepdims=True)
        hit = jnp.min(jnp.where(work == m, rows, NUM_EXPERTS),
                      axis=0, keepdims=True)
        is_hit = rows == hit
        selected = jnp.logical_or(selected, is_hit)
        work = jnp.where(is_hit, _NEG_INF, work)
        tops.append(m)
        idxs.append(hit)
    top = jnp.concatenate(tops, axis=0)
    tidx = jnp.concatenate(idxs, axis=0)

    e = jnp.exp(top - top[0:1, :])
    scores_ref[...] = e / jnp.sum(e, axis=0, keepdims=True)
    idx_ref[...] = tidx

    blk_cnt = jnp.sum(selected.astype(jnp.float32), axis=1, keepdims=True)

    @pl.when(pl.program_id(0) == 0)
    def _init():
        cnt_ref[...] = jnp.zeros_like(cnt_ref)

    cnt_ref[...] += blk_cnt


def kernel(x, weight, bias):
    t = x.shape[0]
    grid = t // BLOCK_T
    bias2 = bias.reshape(NUM_EXPERTS, 1)

    scores_t, tidx_t, cnt = pl.pallas_call(
        _router_kernel,
        grid=(grid,),
        in_specs=[
            pl.BlockSpec((BLOCK_T, HIDDEN), lambda i: (i, 0)),
            pl.BlockSpec((NUM_EXPERTS, HIDDEN), lambda i: (0, 0)),
            pl.BlockSpec((NUM_EXPERTS, 1), lambda i: (0, 0)),
        ],
        out_specs=[
            pl.BlockSpec((TOP_K, BLOCK_T), lambda i: (0, i)),
            pl.BlockSpec((TOP_K, BLOCK_T), lambda i: (0, i)),
            pl.BlockSpec((NUM_EXPERTS, 1), lambda i: (0, 0)),
        ],
        out_shape=[
            jax.ShapeDtypeStruct((TOP_K, t), jnp.float32),
            jax.ShapeDtypeStruct((TOP_K, t), jnp.int32),
            jax.ShapeDtypeStruct((NUM_EXPERTS, 1), jnp.float32),
        ],
    )(x, weight, bias2)
    return scores_t.T, tidx_t.T, cnt.reshape(NUM_EXPERTS)


# speedup vs baseline: 4.9426x; 1.0791x over previous
import jax
import jax.numpy as jnp
from jax.experimental import pallas as pl
from jax.experimental.pallas import tpu as pltpu

NUM_EXPERTS = 64
TOP_K = 8
HIDDEN = 768
BLOCK_T = 4096

_NEG_INF = float("-inf")


def _router_kernel(x_ref, w_ref, bias_ref, scores_ref, idx_ref, cnt_ref):
    logits = jax.lax.dot_general(
        w_ref[...], x_ref[...],
        dimension_numbers=(((1,), (1,)), ((), ())),
        preferred_element_type=jnp.float32,
    )
    logits = logits + bias_ref[...]

    rows = jax.lax.broadcasted_iota(jnp.int32, logits.shape, 0)
    work = logits
    selected = jnp.zeros(logits.shape, dtype=jnp.bool_)
    tops = []
    idxs = []
    for _ in range(TOP_K):
        m = jnp.max(work, axis=0, keepdims=True)
        hit = jnp.min(jnp.where(work == m, rows, NUM_EXPERTS),
                      axis=0, keepdims=True)
        is_hit = rows == hit
        selected = jnp.logical_or(selected, is_hit)
        work = jnp.where(is_hit, _NEG_INF, work)
        tops.append(m)
        idxs.append(hit)
    top = jnp.concatenate(tops, axis=0)
    tidx = jnp.concatenate(idxs, axis=0)

    e = jnp.exp(top - top[0:1, :])
    scores_ref[...] = e / jnp.sum(e, axis=0, keepdims=True)
    idx_ref[...] = tidx

    cnt_ref[...] = jnp.sum(selected.astype(jnp.float32), axis=1,
                           keepdims=True)[None]


def kernel(x, weight, bias):
    t = x.shape[0]
    grid = t // BLOCK_T
    bias2 = bias.reshape(NUM_EXPERTS, 1)

    scores_t, tidx_t, cnt = pl.pallas_call(
        _router_kernel,
        grid=(grid,),
        in_specs=[
            pl.BlockSpec((BLOCK_T, HIDDEN), lambda i: (i, 0)),
            pl.BlockSpec((NUM_EXPERTS, HIDDEN), lambda i: (0, 0)),
            pl.BlockSpec((NUM_EXPERTS, 1), lambda i: (0, 0)),
        ],
        out_specs=[
            pl.BlockSpec((TOP_K, BLOCK_T), lambda i: (0, i)),
            pl.BlockSpec((TOP_K, BLOCK_T), lambda i: (0, i)),
            pl.BlockSpec((1, NUM_EXPERTS, 1), lambda i: (i, 0, 0)),
        ],
        out_shape=[
            jax.ShapeDtypeStruct((TOP_K, t), jnp.float32),
            jax.ShapeDtypeStruct((TOP_K, t), jnp.int32),
            jax.ShapeDtypeStruct((grid, NUM_EXPERTS, 1), jnp.float32),
        ],
        compiler_params=pltpu.CompilerParams(
            dimension_semantics=("parallel",),
        ),
    )(x, weight, bias2)
    return scores_t.T, tidx_t.T, jnp.sum(cnt[:, :, 0], axis=0)
